# baseline (speedup 1.0000x reference)
import functools

import jax
import jax.numpy as jnp
from jax import lax
from jax.experimental import pallas as pl
from jax.experimental.pallas import tpu as pltpu
from jax.experimental.pallas import tpu_sc as plsc

VOCAB = 100000
EMBED = 16
BATCH = 1024
CTX = 20

_NC, _NS = 2, 16
_NW = _NC * _NS
_ROWS_W = BATCH // _NW
_IDX_W = _ROWS_W * CTX
_CHUNK = 128
_NCHUNK = _IDX_W // _CHUNK


def _sc_gather_sum_body(idx_hbm, table_hbm, out_hbm, idx_v, rows_v, out_v, sem):
    wid = lax.axis_index("s") * _NC + lax.axis_index("c")
    pltpu.sync_copy(idx_hbm.at[pl.ds(wid * _IDX_W, _IDX_W)], idx_v)
    copies = [
        pltpu.async_copy(
            table_hbm.at[idx_v.at[pl.ds(c * _CHUNK, _CHUNK)]],
            rows_v.at[pl.ds(c * _CHUNK, _CHUNK)],
            sem,
        )
        for c in range(_NCHUNK)
    ]
    for cp in copies:
        cp.wait()

    def body(r, carry):
        acc = rows_v[r * CTX, :]
        for c in range(1, CTX):
            acc = acc + rows_v[r * CTX + c, :]
        out_v[r, :] = acc
        return carry

    lax.fori_loop(0, _ROWS_W, body, 0)
    pltpu.sync_copy(out_v, out_hbm.at[pl.ds(wid * _ROWS_W, _ROWS_W)])


@functools.lru_cache(maxsize=None)
def _make_sc_gather_sum():
    return pl.kernel(
        _sc_gather_sum_body,
        out_type=jax.ShapeDtypeStruct((BATCH, EMBED), jnp.float32),
        mesh=plsc.VectorSubcoreMesh(core_axis_name="c", subcore_axis_name="s"),
        scratch_types=[
            pltpu.VMEM((_IDX_W,), jnp.int32),
            pltpu.VMEM((_IDX_W, EMBED), jnp.float32),
            pltpu.VMEM((_ROWS_W, EMBED), jnp.float32),
            pltpu.SemaphoreType.DMA,
        ],
        compiler_params=pltpu.CompilerParams(use_tc_tiling_on_sc=False),
    )


_VB = 4096
_NV = (VOCAB + _VB - 1) // _VB

_DN = (((1,), (1,)), ((), ()))


def _stats_body(emb_ref, w_ref, b_ref, z_ref, m_sc, s_sc):
    j = pl.program_id(0)

    @pl.when(j == 0)
    def _():
        m_sc[...] = jnp.full_like(m_sc, -jnp.inf)
        s_sc[...] = jnp.zeros_like(s_sc)

    raw = lax.dot_general(
        emb_ref[...], w_ref[...], _DN, preferred_element_type=jnp.float32
    ) + b_ref[...].reshape(1, _VB)
    col = j * _VB + lax.broadcasted_iota(jnp.int32, (1, _VB), 1)
    logits = jnp.where(col < VOCAB, raw, -jnp.inf)

    m_old = m_sc[...]
    m_new = jnp.maximum(m_old, jnp.max(logits, axis=1, keepdims=True))
    s_new = s_sc[...] * jnp.exp(m_old - m_new) + jnp.sum(
        jnp.exp(logits - m_new), axis=1, keepdims=True
    )
    m_sc[...] = m_new
    s_sc[...] = s_new

    @pl.when(j == _NV - 1)
    def _():
        z_ref[...] = m_new + jnp.log(s_new)


def _write_body(emb_ref, w_ref, b_ref, z_ref, out_ref):
    logits = lax.dot_general(
        emb_ref[...], w_ref[...], _DN, preferred_element_type=jnp.float32
    ) + b_ref[...].reshape(1, _VB)
    out_ref[...] = logits - z_ref[...]


def _tc_logsoftmax(embeds, w, b, interpret=False):
    z = pl.pallas_call(
        _stats_body,
        grid=(_NV,),
        in_specs=[
            pl.BlockSpec((BATCH, EMBED), lambda j: (0, 0)),
            pl.BlockSpec((_VB, EMBED), lambda j: (j, 0)),
            pl.BlockSpec((_VB,), lambda j: (j,)),
        ],
        out_specs=pl.BlockSpec((BATCH, 1), lambda j: (0, 0)),
        out_shape=jax.ShapeDtypeStruct((BATCH, 1), jnp.float32),
        scratch_shapes=[
            pltpu.VMEM((BATCH, 1), jnp.float32),
            pltpu.VMEM((BATCH, 1), jnp.float32),
        ],
        interpret=interpret,
    )(embeds, w, b)
    out = pl.pallas_call(
        _write_body,
        grid=(_NV,),
        in_specs=[
            pl.BlockSpec((BATCH, EMBED), lambda j: (0, 0)),
            pl.BlockSpec((_VB, EMBED), lambda j: (j, 0)),
            pl.BlockSpec((_VB,), lambda j: (j,)),
            pl.BlockSpec((BATCH, 1), lambda j: (0, 0)),
        ],
        out_specs=pl.BlockSpec((BATCH, _VB), lambda j: (0, j)),
        out_shape=jax.ShapeDtypeStruct((BATCH, VOCAB), jnp.float32),
        interpret=interpret,
    )(embeds, w, b, z)
    return out


def kernel(inputs, emb_table, W, b):
    idx = inputs.reshape(BATCH * CTX).astype(jnp.int32)
    embeds = _make_sc_gather_sum()(idx, emb_table)
    w_bf = W.astype(jnp.bfloat16)
    return _tc_logsoftmax(embeds.astype(jnp.bfloat16), w_bf, b)


# speedup vs baseline: 2.0043x; 2.0043x over previous
import functools

import jax
import jax.numpy as jnp
from jax import lax
from jax.experimental import pallas as pl
from jax.experimental.pallas import tpu as pltpu
from jax.experimental.pallas import tpu_sc as plsc

VOCAB = 100000
EMBED = 16
BATCH = 1024
CTX = 20

_NC, _NS = 2, 16
_NW = _NC * _NS
_ROWS_W = BATCH // _NW
_IDX_W = _ROWS_W * CTX
_CHUNK = 128
_NCHUNK = _IDX_W // _CHUNK


def _sc_gather_sum_body(idx_hbm, table_hbm, out_hbm, idx_v, rows_v, out_v, sem):
    wid = lax.axis_index("s") * _NC + lax.axis_index("c")
    pltpu.sync_copy(idx_hbm.at[pl.ds(wid * _IDX_W, _IDX_W)], idx_v)
    copies = [
        pltpu.async_copy(
            table_hbm.at[idx_v.at[pl.ds(c * _CHUNK, _CHUNK)]],
            rows_v.at[pl.ds(c * _CHUNK, _CHUNK)],
            sem,
        )
        for c in range(_NCHUNK)
    ]
    for cp in copies:
        cp.wait()

    def body(r, carry):
        acc = rows_v[r * CTX, :]
        for c in range(1, CTX):
            acc = acc + rows_v[r * CTX + c, :]
        out_v[r, :] = acc
        return carry

    lax.fori_loop(0, _ROWS_W, body, 0)
    pltpu.sync_copy(out_v, out_hbm.at[pl.ds(wid * _ROWS_W, _ROWS_W)])


@functools.lru_cache(maxsize=None)
def _make_sc_gather_sum():
    return pl.kernel(
        _sc_gather_sum_body,
        out_type=jax.ShapeDtypeStruct((BATCH, EMBED), jnp.float32),
        mesh=plsc.VectorSubcoreMesh(core_axis_name="c", subcore_axis_name="s"),
        scratch_types=[
            pltpu.VMEM((_IDX_W,), jnp.int32),
            pltpu.VMEM((_IDX_W, EMBED), jnp.float32),
            pltpu.VMEM((_ROWS_W, EMBED), jnp.float32),
            pltpu.SemaphoreType.DMA,
        ],
        compiler_params=pltpu.CompilerParams(use_tc_tiling_on_sc=False),
    )


_VB = 4096
_NV = (VOCAB + _VB - 1) // _VB
_K = EMBED + 1

_DN_T = (((0,), (1,)), ((), ()))


def _stats_body(wt_ref, emb_ref, z_ref, m_sc, s_sc):
    j = pl.program_id(0)

    @pl.when(j == 0)
    def _():
        m_sc[...] = jnp.full_like(m_sc, -jnp.inf)
        s_sc[...] = jnp.zeros_like(s_sc)

    raw = lax.dot_general(
        wt_ref[...], emb_ref[...], _DN_T, preferred_element_type=jnp.float32
    )
    row = j * _VB + lax.broadcasted_iota(jnp.int32, (_VB, 1), 0)
    logits = jnp.where(row < VOCAB, raw, -jnp.inf)

    m_old = m_sc[...]
    m_new = jnp.maximum(m_old, jnp.max(logits, axis=0, keepdims=True))
    s_new = s_sc[...] * jnp.exp(m_old - m_new) + jnp.sum(
        jnp.exp(logits - m_new), axis=0, keepdims=True
    )
    m_sc[...] = m_new
    s_sc[...] = s_new

    @pl.when(j == _NV - 1)
    def _():
        z_ref[...] = m_new + jnp.log(s_new)


def _write_body(wt_ref, emb_ref, z_ref, out_ref):
    logits = lax.dot_general(
        wt_ref[...], emb_ref[...], _DN_T, preferred_element_type=jnp.float32
    )
    out_ref[...] = logits - z_ref[...]


def _tc_logsoftmax_t(wt_ext, emb_ext, interpret=False):
    z = pl.pallas_call(
        _stats_body,
        grid=(_NV,),
        in_specs=[
            pl.BlockSpec((_K, _VB), lambda j: (0, j)),
            pl.BlockSpec((BATCH, _K), lambda j: (0, 0)),
        ],
        out_specs=pl.BlockSpec((1, BATCH), lambda j: (0, 0)),
        out_shape=jax.ShapeDtypeStruct((1, BATCH), jnp.float32),
        scratch_shapes=[
            pltpu.VMEM((1, BATCH), jnp.float32),
            pltpu.VMEM((1, BATCH), jnp.float32),
        ],
        interpret=interpret,
    )(wt_ext, emb_ext)
    out_t = pl.pallas_call(
        _write_body,
        grid=(_NV,),
        in_specs=[
            pl.BlockSpec((_K, _VB), lambda j: (0, j)),
            pl.BlockSpec((BATCH, _K), lambda j: (0, 0)),
            pl.BlockSpec((1, BATCH), lambda j: (0, 0)),
        ],
        out_specs=pl.BlockSpec((_VB, BATCH), lambda j: (j, 0)),
        out_shape=jax.ShapeDtypeStruct((VOCAB, BATCH), jnp.float32),
        interpret=interpret,
    )(wt_ext, emb_ext, z)
    return out_t


def kernel(inputs, emb_table, W, b):
    idx = inputs.reshape(BATCH * CTX).astype(jnp.int32)
    embeds = _make_sc_gather_sum()(idx, emb_table)
    wt_ext = jnp.concatenate(
        [W.T.astype(jnp.bfloat16), b.astype(jnp.bfloat16).reshape(1, VOCAB)],
        axis=0,
    )
    emb_ext = jnp.concatenate(
        [embeds.astype(jnp.bfloat16), jnp.ones((BATCH, 1), jnp.bfloat16)],
        axis=1,
    )
    out_t = _tc_logsoftmax_t(wt_ext, emb_ext)
    return out_t.T


# speedup vs baseline: 2.1394x; 1.0674x over previous
import functools

import jax
import jax.numpy as jnp
from jax import lax
from jax.experimental import pallas as pl
from jax.experimental.pallas import tpu as pltpu
from jax.experimental.pallas import tpu_sc as plsc

VOCAB = 100000
EMBED = 16
BATCH = 1024
CTX = 20

_NC, _NS = 2, 16
_NW = _NC * _NS
_ROWS_W = BATCH // _NW
_IDX_W = _ROWS_W * CTX
_CHUNK = 128
_NCHUNK = _IDX_W // _CHUNK


def _sc_gather_sum_body(idx_hbm, table_hbm, out_hbm, idx_v, rows_v, out_v, sem):
    wid = lax.axis_index("s") * _NC + lax.axis_index("c")
    pltpu.sync_copy(idx_hbm.at[pl.ds(wid * _IDX_W, _IDX_W)], idx_v)
    copies = [
        pltpu.async_copy(
            table_hbm.at[idx_v.at[pl.ds(c * _CHUNK, _CHUNK)]],
            rows_v.at[pl.ds(c * _CHUNK, _CHUNK)],
            sem,
        )
        for c in range(_NCHUNK)
    ]
    for cp in copies:
        cp.wait()

    def body(r, carry):
        acc = rows_v[r * CTX, :]
        for c in range(1, CTX):
            acc = acc + rows_v[r * CTX + c, :]
        out_v[r, :] = acc
        return carry

    lax.fori_loop(0, _ROWS_W, body, 0)
    pltpu.sync_copy(out_v, out_hbm.at[pl.ds(wid * _ROWS_W, _ROWS_W)])


@functools.lru_cache(maxsize=None)
def _make_sc_gather_sum():
    return pl.kernel(
        _sc_gather_sum_body,
        out_type=jax.ShapeDtypeStruct((BATCH, EMBED), jnp.float32),
        mesh=plsc.VectorSubcoreMesh(core_axis_name="c", subcore_axis_name="s"),
        scratch_types=[
            pltpu.VMEM((_IDX_W,), jnp.int32),
            pltpu.VMEM((_IDX_W, EMBED), jnp.float32),
            pltpu.VMEM((_ROWS_W, EMBED), jnp.float32),
            pltpu.SemaphoreType.DMA,
        ],
        compiler_params=pltpu.CompilerParams(use_tc_tiling_on_sc=False),
    )


_VB = 4096
_NV = (VOCAB + _VB - 1) // _VB
_VPAD = _NV * _VB
_K = EMBED + 1

_DN_T = (((0,), (1,)), ((), ()))


def _stats_body(wt_ref, emb_ref, z_ref, m_sc, s_sc):
    j = pl.program_id(0)

    @pl.when(j == 0)
    def _():
        m_sc[...] = jnp.full_like(m_sc, -jnp.inf)
        s_sc[...] = jnp.zeros_like(s_sc)

    logits = lax.dot_general(
        wt_ref[...], emb_ref[...], _DN_T, preferred_element_type=jnp.float32
    )

    m_old = m_sc[...]
    m_new = jnp.maximum(m_old, jnp.max(logits, axis=0, keepdims=True))
    s_new = s_sc[...] * jnp.exp(m_old - m_new) + jnp.sum(
        jnp.exp(logits - m_new), axis=0, keepdims=True
    )
    m_sc[...] = m_new
    s_sc[...] = s_new

    @pl.when(j == _NV - 1)
    def _():
        z_ref[...] = m_new + jnp.log(s_new)


def _write_body(wt_ref, emb_ref, z_ref, out_ref):
    logits = lax.dot_general(
        wt_ref[...], emb_ref[...], _DN_T, preferred_element_type=jnp.float32
    )
    out_ref[...] = logits - z_ref[...]


def _tc_logsoftmax_t(wt_ext, emb_ext, interpret=False):
    z = pl.pallas_call(
        _stats_body,
        grid=(_NV,),
        in_specs=[
            pl.BlockSpec((_K, _VB), lambda j: (0, j)),
            pl.BlockSpec((BATCH, _K), lambda j: (0, 0)),
        ],
        out_specs=pl.BlockSpec((1, BATCH), lambda j: (0, 0)),
        out_shape=jax.ShapeDtypeStruct((1, BATCH), jnp.float32),
        scratch_shapes=[
            pltpu.VMEM((1, BATCH), jnp.float32),
            pltpu.VMEM((1, BATCH), jnp.float32),
        ],
        interpret=interpret,
    )(wt_ext, emb_ext)
    out_t = pl.pallas_call(
        _write_body,
        grid=(_NV,),
        in_specs=[
            pl.BlockSpec((_K, _VB), lambda j: (0, j)),
            pl.BlockSpec((BATCH, _K), lambda j: (0, 0)),
            pl.BlockSpec((1, BATCH), lambda j: (0, 0)),
        ],
        out_specs=pl.BlockSpec((_VB, BATCH), lambda j: (j, 0)),
        out_shape=jax.ShapeDtypeStruct((VOCAB, BATCH), jnp.float32),
        interpret=interpret,
    )(wt_ext, emb_ext, z)
    return out_t


def kernel(inputs, emb_table, W, b):
    idx = inputs.reshape(BATCH * CTX).astype(jnp.int32)
    embeds = _make_sc_gather_sum()(idx, emb_table)
    wt_pad = jnp.pad(W.T.astype(jnp.bfloat16), ((0, 0), (0, _VPAD - VOCAB)))
    b_pad = jnp.pad(
        b.astype(jnp.bfloat16).reshape(1, VOCAB),
        ((0, 0), (0, _VPAD - VOCAB)),
        constant_values=jnp.bfloat16(-1e30),
    )
    wt_ext = jnp.concatenate([wt_pad, b_pad], axis=0)
    emb_ext = jnp.concatenate(
        [embeds.astype(jnp.bfloat16), jnp.ones((BATCH, 1), jnp.bfloat16)],
        axis=1,
    )
    out_t = _tc_logsoftmax_t(wt_ext, emb_ext)
    return out_t.T


# speedup vs baseline: 2.2639x; 1.0582x over previous
import functools

import jax
import jax.numpy as jnp
from jax import lax
from jax.experimental import pallas as pl
from jax.experimental.pallas import tpu as pltpu
from jax.experimental.pallas import tpu_sc as plsc

VOCAB = 100000
EMBED = 16
BATCH = 1024
CTX = 20

_NC, _NS = 2, 16
_NW = _NC * _NS
_ROWS_W = BATCH // _NW
_IDX_W = _ROWS_W * CTX
_CHUNK = 128
_NCHUNK = _IDX_W // _CHUNK


def _sc_gather_sum_body(idx_hbm, table_hbm, out_hbm, idx_v, rows_v, out_v, sem):
    wid = lax.axis_index("s") * _NC + lax.axis_index("c")
    pltpu.sync_copy(idx_hbm.at[pl.ds(wid * _IDX_W, _IDX_W)], idx_v)
    copies = [
        pltpu.async_copy(
            table_hbm.at[idx_v.at[pl.ds(c * _CHUNK, _CHUNK)]],
            rows_v.at[pl.ds(c * _CHUNK, _CHUNK)],
            sem,
        )
        for c in range(_NCHUNK)
    ]
    for cp in copies:
        cp.wait()

    def body(r, carry):
        acc = rows_v[r * CTX, :]
        for c in range(1, CTX):
            acc = acc + rows_v[r * CTX + c, :]
        out_v[r, :] = acc
        return carry

    lax.fori_loop(0, _ROWS_W, body, 0)
    pltpu.sync_copy(out_v, out_hbm.at[pl.ds(wid * _ROWS_W, _ROWS_W)])


@functools.lru_cache(maxsize=None)
def _make_sc_gather_sum():
    return pl.kernel(
        _sc_gather_sum_body,
        out_type=jax.ShapeDtypeStruct((BATCH, EMBED), jnp.float32),
        mesh=plsc.VectorSubcoreMesh(core_axis_name="c", subcore_axis_name="s"),
        scratch_types=[
            pltpu.VMEM((_IDX_W,), jnp.int32),
            pltpu.VMEM((_IDX_W, EMBED), jnp.float32),
            pltpu.VMEM((_ROWS_W, EMBED), jnp.float32),
            pltpu.SemaphoreType.DMA,
        ],
        compiler_params=pltpu.CompilerParams(use_tc_tiling_on_sc=False),
    )


_VB = 4096
_NV = (VOCAB + _VB - 1) // _VB
_VPAD = _NV * _VB
_K = EMBED + 1

_DN_T = (((0,), (1,)), ((), ()))


_NCH = 4
_CB = BATCH // _NCH


def _fused_body(wt_ref, emb_cur_ref, emb_prev_ref, out_ref, m_sc, s_sc, z_sc):
    q = pl.program_id(0)
    j = pl.program_id(1)

    @pl.when(q >= 1)
    def _():
        logits = lax.dot_general(
            wt_ref[...], emb_prev_ref[...], _DN_T,
            preferred_element_type=jnp.float32,
        )
        out_ref[...] = logits - z_sc[...]

    @pl.when(q < _NCH)
    def _():
        @pl.when(j == 0)
        def _():
            m_sc[...] = jnp.full_like(m_sc, -jnp.inf)
            s_sc[...] = jnp.zeros_like(s_sc)

        logits = lax.dot_general(
            wt_ref[...], emb_cur_ref[...], _DN_T,
            preferred_element_type=jnp.float32,
        )
        m_old = m_sc[...]
        m_new = jnp.maximum(m_old, jnp.max(logits, axis=0, keepdims=True))
        s_new = s_sc[...] * jnp.exp(m_old - m_new) + jnp.sum(
            jnp.exp(logits - m_new), axis=0, keepdims=True
        )
        m_sc[...] = m_new
        s_sc[...] = s_new

        @pl.when(j == _NV - 1)
        def _():
            z_sc[...] = m_new + jnp.log(s_new)


def _tc_logsoftmax_t(wt_ext, emb_ext, interpret=False):
    out_t = pl.pallas_call(
        _fused_body,
        grid=(_NCH + 1, _NV),
        in_specs=[
            pl.BlockSpec((_K, _VB), lambda q, j: (0, j)),
            pl.BlockSpec((_CB, _K), lambda q, j: (jnp.minimum(q, _NCH - 1), 0)),
            pl.BlockSpec((_CB, _K), lambda q, j: (jnp.maximum(q - 1, 0), 0)),
        ],
        out_specs=pl.BlockSpec(
            (_VB, _CB),
            lambda q, j: (jnp.where(q >= 1, j, 0), jnp.maximum(q - 1, 0)),
        ),
        out_shape=jax.ShapeDtypeStruct((VOCAB, BATCH), jnp.float32),
        scratch_shapes=[
            pltpu.VMEM((1, _CB), jnp.float32),
            pltpu.VMEM((1, _CB), jnp.float32),
            pltpu.VMEM((1, _CB), jnp.float32),
        ],
        interpret=interpret,
    )(wt_ext, emb_ext, emb_ext)
    return out_t


def kernel(inputs, emb_table, W, b):
    idx = inputs.reshape(BATCH * CTX).astype(jnp.int32)
    embeds = _make_sc_gather_sum()(idx, emb_table)
    wt_pad = jnp.pad(W.T.astype(jnp.bfloat16), ((0, 0), (0, _VPAD - VOCAB)))
    b_pad = jnp.pad(
        b.astype(jnp.bfloat16).reshape(1, VOCAB),
        ((0, 0), (0, _VPAD - VOCAB)),
        constant_values=jnp.bfloat16(-1e30),
    )
    wt_ext = jnp.concatenate([wt_pad, b_pad], axis=0)
    emb_ext = jnp.concatenate(
        [embeds.astype(jnp.bfloat16), jnp.ones((BATCH, 1), jnp.bfloat16)],
        axis=1,
    )
    out_t = _tc_logsoftmax_t(wt_ext, emb_ext)
    return out_t.T
